# Initial kernel scaffold; baseline (speedup 1.0000x reference)
import jax
import jax.numpy as jnp
from jax.experimental import pallas as pl


def kernel(inputs, sig, W1, W2, Wfc, bfc):
    raise NotImplementedError("write your pallas kernel here")


# speedup vs baseline: 44.5536x; 44.5536x over previous
import jax
import jax.numpy as jnp
from jax.experimental import pallas as pl

_B, _N, _K, _D, _C1, _C2, _NC = 32, 512, 20, 5, 30, 40, 40
_F32 = jnp.float32


def _norm_over_d(acc, c):
    s = None
    for d in range(_D):
        sl = acc[:, d * c:(d + 1) * c]
        s = sl * sl if s is None else s + sl * sl
    denom = jnp.sqrt(s) + 1e-8
    return jnp.concatenate([acc[:, d * c:(d + 1) * c] / denom for d in range(_D)], axis=1)


def _dist_topk_wfm(F, wblk_ref, dout):
    n = F.shape[0]
    sq = jnp.sum(F * F, axis=1, keepdims=True)
    G = jax.lax.dot_general(F, F, (((1,), (1,)), ((), ())),
                            preferred_element_type=_F32)
    adj = sq + sq.reshape(1, n) - 2.0 * G
    col = jax.lax.broadcasted_iota(jnp.int32, (n, n), 1)

    def body(k, carry):
        adj, acc = carry
        m = jnp.min(adj, axis=1, keepdims=True)
        ismin = adj == m
        idx = jnp.min(jnp.where(ismin, col, n), axis=1)
        chosen = col == idx[:, None]
        onehot = chosen.astype(_F32)
        nb = jax.lax.dot_general(onehot, F, (((1,), (0,)), ((), ())),
                                 preferred_element_type=_F32)
        wk = wblk_ref[k]
        acc = acc + jax.lax.dot_general(nb, wk, (((1,), (0,)), ((), ())),
                                        preferred_element_type=_F32)
        adj = jnp.where(chosen, 1e30, adj)
        return adj, acc

    _, acc = jax.lax.fori_loop(
        0, _K, body, (adj, jnp.zeros((n, dout), _F32)))
    return acc


def _net_body(x_ref, sig_ref, w1_ref, w2_ref, dist_ref):
    x3 = x_ref[0]
    sigc = sig_ref[...]
    norms = jnp.sqrt(jnp.sum(x3 * x3, axis=1, keepdims=True) + 1e-8)
    feat = jnp.concatenate([x3, norms * sigc, jnp.ones_like(norms)], axis=1)
    feat = feat / (jnp.sqrt(jnp.sum(feat * feat, axis=1, keepdims=True)) + 1e-8)

    acc1 = _dist_topk_wfm(feat, w1_ref, _D * _C1)
    fm1 = _norm_over_d(acc1, _C1)
    acc2 = _dist_topk_wfm(fm1, w2_ref, _D * _C2)
    fm2 = _norm_over_d(acc2, _C2)

    m = jnp.mean(fm2, axis=0, keepdims=True)
    diff = fm2 - m
    s = None
    for d in range(_D):
        sl = diff[:, d * _C2:(d + 1) * _C2]
        s = sl * sl if s is None else s + sl * sl
    dist_ref[0] = jnp.sqrt(s + 1e-8)


def _fc_body(a_ref, w_ref, b_ref, o_ref):
    o_ref[...] = jax.lax.dot_general(
        a_ref[...], w_ref[...], (((1,), (0,)), ((), ())),
        preferred_element_type=_F32) + b_ref[...]


def kernel(inputs, sig, W1, W2, Wfc, bfc):
    w1s = jax.nn.softmax(W1, axis=0)
    w2s = jax.nn.softmax(W2, axis=0).reshape(_K, _C1, _C2)
    eye = jnp.eye(_D, dtype=_F32)
    w1blk = jnp.einsum('kc,de->kdec', w1s, eye).reshape(_K, _D, _D * _C1)
    w2blk = jnp.einsum('kco,de->kdceo', w2s, eye).reshape(_K, _D * _C1, _D * _C2)

    dist = pl.pallas_call(
        _net_body,
        grid=(_B,),
        in_specs=[
            pl.BlockSpec((1, _N, 3), lambda b: (b, 0, 0)),
            pl.BlockSpec((_N, 1), lambda b: (0, 0)),
            pl.BlockSpec((_K, _D, _D * _C1), lambda b: (0, 0, 0)),
            pl.BlockSpec((_K, _D * _C1, _D * _C2), lambda b: (0, 0, 0)),
        ],
        out_specs=pl.BlockSpec((1, _N, _C2), lambda b: (b, 0, 0)),
        out_shape=jax.ShapeDtypeStruct((_B, _N, _C2), _F32),
    )(inputs, sig.reshape(_N, 1), w1blk, w2blk)

    flat = dist.reshape(_B, _N * _C2)
    out = pl.pallas_call(
        _fc_body,
        in_specs=[
            pl.BlockSpec((_B, _N * _C2), lambda: (0, 0)),
            pl.BlockSpec((_N * _C2, _NC), lambda: (0, 0)),
            pl.BlockSpec((1, _NC), lambda: (0, 0)),
        ],
        out_specs=pl.BlockSpec((_B, _NC), lambda: (0, 0)),
        out_shape=jax.ShapeDtypeStruct((_B, _NC), _F32),
    )(flat, Wfc, bfc.reshape(1, _NC))
    return out


# speedup vs baseline: 91.7193x; 2.0586x over previous
import jax
import jax.numpy as jnp
from jax.experimental import pallas as pl

_B, _N, _K, _D, _C1, _C2, _NC = 32, 512, 20, 5, 30, 40, 40
_F32 = jnp.float32


def _mm(a, b):
    return jax.lax.dot_general(a, b, (((1,), (0,)), ((), ())),
                               preferred_element_type=_F32)


def _dist_topk_wfm(F, wblk_ref, msum_ref, mdup_ref, dout):
    n = F.shape[0]
    sq = jnp.sum(F * F, axis=1, keepdims=True)
    G = jax.lax.dot_general(F, F, (((1,), (1,)), ((), ())),
                            preferred_element_type=_F32)
    adj = sq + sq.reshape(1, n) - 2.0 * G
    col = jax.lax.broadcasted_iota(jnp.int32, (n, n), 1)

    acc = jnp.zeros((n, dout), _F32)
    for k in range(_K):
        m = jnp.min(adj, axis=1, keepdims=True)
        idx = jnp.min(jnp.where(adj == m, col, n), axis=1)
        onehot = (col == idx[:, None]).astype(_F32)
        nb = _mm(onehot, F)
        acc = acc + _mm(nb, wblk_ref[k])
        adj = adj + onehot * 1e30
    s = _mm(acc * acc, msum_ref[...])
    denom = jnp.sqrt(s) + 1e-8
    return acc / _mm(denom, mdup_ref[...])


def _net_body(x_ref, sig_ref, w1_ref, w2_ref, ms1_ref, md1_ref,
              ms2_ref, md2_ref, dist_ref):
    x3 = x_ref[0]
    sigc = sig_ref[...]
    norms = jnp.sqrt(jnp.sum(x3 * x3, axis=1, keepdims=True) + 1e-8)
    feat = jnp.concatenate([x3, norms * sigc, jnp.ones_like(norms)], axis=1)
    feat = feat / (jnp.sqrt(jnp.sum(feat * feat, axis=1, keepdims=True)) + 1e-8)

    fm1 = _dist_topk_wfm(feat, w1_ref, ms1_ref, md1_ref, _D * _C1)
    fm2 = _dist_topk_wfm(fm1, w2_ref, ms2_ref, md2_ref, _D * _C2)

    m = _mm(jnp.full((1, _N), 1.0 / _N, _F32), fm2)
    diff = fm2 - m
    s = _mm(diff * diff, ms2_ref[...])
    dist_ref[0] = jnp.sqrt(s + 1e-8)


def _fc_body(a_ref, w_ref, b_ref, o_ref):
    o_ref[...] = jax.lax.dot_general(
        a_ref[...], w_ref[...], (((1,), (0,)), ((), ())),
        preferred_element_type=_F32) + b_ref[...]


def kernel(inputs, sig, W1, W2, Wfc, bfc):
    w1s = jax.nn.softmax(W1, axis=0)
    w2s = jax.nn.softmax(W2, axis=0).reshape(_K, _C1, _C2)
    eye = jnp.eye(_D, dtype=_F32)
    w1blk = jnp.einsum('kc,de->kdec', w1s, eye).reshape(_K, _D, _D * _C1)
    w2blk = jnp.einsum('kco,de->kdceo', w2s, eye).reshape(_K, _D * _C1, _D * _C2)
    ms1 = jnp.tile(jnp.eye(_C1, dtype=_F32), (_D, 1))
    ms2 = jnp.tile(jnp.eye(_C2, dtype=_F32), (_D, 1))

    cspec = lambda shape: pl.BlockSpec(shape, lambda b: tuple(0 for _ in shape))
    dist = pl.pallas_call(
        _net_body,
        grid=(_B,),
        in_specs=[
            pl.BlockSpec((1, _N, 3), lambda b: (b, 0, 0)),
            cspec((_N, 1)),
            cspec((_K, _D, _D * _C1)),
            cspec((_K, _D * _C1, _D * _C2)),
            cspec((_D * _C1, _C1)),
            cspec((_C1, _D * _C1)),
            cspec((_D * _C2, _C2)),
            cspec((_C2, _D * _C2)),
        ],
        out_specs=pl.BlockSpec((1, _N, _C2), lambda b: (b, 0, 0)),
        out_shape=jax.ShapeDtypeStruct((_B, _N, _C2), _F32),
    )(inputs, sig.reshape(_N, 1), w1blk, w2blk,
      ms1, ms1.T, ms2, ms2.T)

    flat = dist.reshape(_B, _N * _C2)
    out = pl.pallas_call(
        _fc_body,
        in_specs=[
            pl.BlockSpec((_B, _N * _C2), lambda: (0, 0)),
            pl.BlockSpec((_N * _C2, _NC), lambda: (0, 0)),
            pl.BlockSpec((1, _NC), lambda: (0, 0)),
        ],
        out_specs=pl.BlockSpec((_B, _NC), lambda: (0, 0)),
        out_shape=jax.ShapeDtypeStruct((_B, _NC), _F32),
    )(flat, Wfc, bfc.reshape(1, _NC))
    return out
